# baseline (speedup 1.0000x reference)
import functools

import numpy as np
import jax
import jax.numpy as jnp
from jax import lax
from jax.experimental import pallas as pl
from jax.experimental.pallas import tpu as pltpu
from jax.experimental.pallas import tpu_sc as plsc

B = 8
NPER = 32768
NQ = 64
K = 32
STEP = NPER // NQ
NTOT = B * NPER
NQTOT = B * NQ
NTILES = 32
QPT = NQTOT // NTILES
TPB = NTILES // B
NCHUNK = NPER // 16
INF = np.float32(3.4e38)


def _lex_lt(ka, ia, kb, ib):
    return (ka < kb) | ((ka == kb) & (ia < ib))


FCAP = 48


def _knn_body(pts, ctr, out, xs, ys, zs, cbuf, bufd, bufi, outv,
              fifod, fifoi, curs):
    cid = lax.axis_index("c")
    sid = lax.axis_index("s")
    wid = sid * 2 + cid
    bi = wid // TPB
    qoff = (wid % TPB) * QPT
    base = bi * NPER

    pltpu.sync_copy(pts.at[pl.ds(base, NPER)], xs)
    pltpu.sync_copy(pts.at[pl.ds(NTOT + base, NPER)], ys)
    pltpu.sync_copy(pts.at[pl.ds(2 * NTOT + base, NPER)], zs)
    qbase = wid * QPT
    for c in range(3):
        pltpu.sync_copy(ctr.at[pl.ds(c * NQTOT + qbase, QPT)],
                        cbuf.at[pl.ds(c * QPT, QPT)])

    inf16 = jnp.full((16,), INF, jnp.float32)
    zero16 = jnp.zeros((16,), jnp.int32)
    for q in range(QPT):
        curs[q] = 0
        for h in range(2):
            bufd[pl.ds(q * K + h * 16, 16)] = inf16
            bufi[pl.ds(q * K + h * 16, 16)] = zero16

    cxv = cbuf[pl.ds(0 * QPT, 16)]
    cyv = cbuf[pl.ds(1 * QPT, 16)]
    czv = cbuf[pl.ds(2 * QPT, 16)]
    cxs = [cxv[q] for q in range(QPT)]
    cys = [cyv[q] for q in range(QPT)]
    czs = [czv[q] for q in range(QPT)]

    iota16 = lax.iota(jnp.int32, 16)

    def _merge(q, dm, ii):
        snew, inew = plsc.sort_key_val(dm, ii)
        b0d = bufd[pl.ds(q * K, 16)]
        b1d = bufd[pl.ds(q * K + 16, 16)]
        b0i = bufi[pl.ds(q * K, 16)]
        b1i = bufi[pl.ds(q * K + 16, 16)]
        rb1d = lax.rev(b1d, (0,))
        rb1i = lax.rev(b1i, (0,))
        lt = _lex_lt(snew, inew, rb1d, rb1i)
        ld = jnp.where(lt, snew, rb1d)
        li = jnp.where(lt, inew, rb1i)
        lsd, lsi = plsc.sort_key_val(ld, li)
        rld = lax.rev(lsd, (0,))
        rli = lax.rev(lsi, (0,))
        lt2 = _lex_lt(b0d, b0i, rld, rli)
        lod = jnp.where(lt2, b0d, rld)
        loi = jnp.where(lt2, b0i, rli)
        hid = jnp.where(lt2, rld, b0d)
        hii = jnp.where(lt2, rli, b0i)
        nb0d, nb0i = plsc.sort_key_val(lod, loi)
        nb1d, nb1i = plsc.sort_key_val(hid, hii)
        bufd[pl.ds(q * K, 16)] = nb0d
        bufd[pl.ds(q * K + 16, 16)] = nb1d
        bufi[pl.ds(q * K, 16)] = nb0i
        bufi[pl.ds(q * K + 16, 16)] = nb1i
        return nb1d[15]

    def _chunk(ci, ts):
        b16 = ci * 16
        px = xs[pl.ds(b16, 16)]
        py = ys[pl.ds(b16, 16)]
        pz = zs[pl.ds(b16, 16)]
        sqs = []
        masks = []
        for q in range(QPT):
            dx = px - cxs[q]
            dy = py - cys[q]
            dz = pz - czs[q]
            sq = dx * dx + dy * dy + dz * dz
            sqs.append(sq)
            masks.append(sq < ts[q])
        anym = masks[0]
        for q in range(1, QPT):
            anym = anym | masks[q]

        def _taken():
            iv = b16 + iota16
            new_ts = []
            for q in range(QPT):
                qb = q * FCAP

                def _append(q=q, qb=qb):
                    m = masks[q]
                    cnt = plsc.all_reduce_population_count(m)[0]
                    cur = curs[q]
                    plsc.store_compressed(
                        fifod.at[pl.ds(qb + cur, 16)], sqs[q], mask=m)
                    plsc.store_compressed(
                        fifoi.at[pl.ds(qb + cur, 16)], iv, mask=m)
                    ncur = cur + cnt

                    def _do_merge(q=q, qb=qb, ncur=ncur):
                        f0d = fifod[pl.ds(qb, 16)]
                        f0i = fifoi[pl.ds(qb, 16)]
                        tnew = _merge(q, f0d, f0i)
                        fifod[pl.ds(qb, 16)] = fifod[pl.ds(qb + 16, 16)]
                        fifoi[pl.ds(qb, 16)] = fifoi[pl.ds(qb + 16, 16)]
                        curs[q] = ncur - 16
                        return tnew

                    def _no_merge(q=q, ncur=ncur):
                        curs[q] = ncur
                        return ts[q]

                    return lax.cond(ncur >= 16, _do_merge, _no_merge)

                new_ts.append(
                    lax.cond(jnp.any(masks[q]), _append, lambda q=q: ts[q]))
            return tuple(new_ts)

        return lax.cond(jnp.any(anym), _taken, lambda: ts)

    ts0 = (INF,) * QPT
    ts = lax.fori_loop(0, NCHUNK, _chunk, ts0)

    for q in range(QPT):
        cur = curs[q]
        f0d = fifod[pl.ds(q * FCAP, 16)]
        f0i = fifoi[pl.ds(q * FCAP, 16)]
        dm = jnp.where(iota16 < cur, f0d, INF)
        _merge(q, dm, f0i)
    del ts

    for q in range(QPT):
        for h in range(2):
            ii = bufi[pl.ds(q * K + h * 16, 16)]
            xg = plsc.load_gather(xs, [ii]) - cxs[q]
            yg = plsc.load_gather(ys, [ii]) - cys[q]
            zg = plsc.load_gather(zs, [ii]) - czs[q]
            outv[pl.ds(0 * QPT * K + q * K + h * 16, 16)] = xg
            outv[pl.ds(1 * QPT * K + q * K + h * 16, 16)] = yg
            outv[pl.ds(2 * QPT * K + q * K + h * 16, 16)] = zg
    obase = wid * QPT * K
    for c in range(3):
        pltpu.sync_copy(outv.at[pl.ds(c * QPT * K, QPT * K)],
                        out.at[pl.ds(c * NQTOT * K + obase, QPT * K)])


@functools.cache
def _knn_kernel():
    return pl.kernel(
        _knn_body,
        out_type=jax.ShapeDtypeStruct((3 * NQTOT * K,), jnp.float32),
        mesh=plsc.VectorSubcoreMesh(core_axis_name="c", subcore_axis_name="s"),
        compiler_params=pltpu.CompilerParams(needs_layout_passes=False),
        scratch_types=[
            pltpu.VMEM((NPER,), jnp.float32),
            pltpu.VMEM((NPER,), jnp.float32),
            pltpu.VMEM((NPER,), jnp.float32),
            pltpu.VMEM((3 * QPT,), jnp.float32),
            pltpu.VMEM((QPT * K,), jnp.float32),
            pltpu.VMEM((QPT * K,), jnp.int32),
            pltpu.VMEM((3 * QPT * K,), jnp.float32),
            pltpu.VMEM((QPT * FCAP,), jnp.float32),
            pltpu.VMEM((QPT * FCAP,), jnp.int32),
            pltpu.SMEM((QPT,), jnp.int32),
        ],
    )


def _knn(pts, ctr):
    return _knn_kernel()(pts, ctr)


def _gelu(x):
    return 0.5 * x * (1.0 + lax.erf(x * jnp.float32(0.7071067811865476)))


def _mlp_body(rel, w1, b1, g1, be1, w2, b2, g2, be2, w3, b3, g3, be3, out):
    for bi in range(B):
        x = rel[bi]
        a = jnp.dot(x, w1[...], preferred_element_type=jnp.float32) + b1[...]
        mu = jnp.mean(a, axis=0, keepdims=True)
        va = jnp.mean((a - mu) * (a - mu), axis=0, keepdims=True)
        a = (a - mu) / jnp.sqrt(va + 1e-5) * g1[...] + be1[...]
        a = _gelu(a)
        a = jnp.dot(a, w2[...], preferred_element_type=jnp.float32) + b2[...]
        mu = jnp.mean(a, axis=0, keepdims=True)
        va = jnp.mean((a - mu) * (a - mu), axis=0, keepdims=True)
        a = (a - mu) / jnp.sqrt(va + 1e-5) * g2[...] + be2[...]
        a = _gelu(a)
        a = jnp.dot(a, w3[...], preferred_element_type=jnp.float32) + b3[...]
        mu = jnp.mean(a, axis=0, keepdims=True)
        va = jnp.mean((a - mu) * (a - mu), axis=0, keepdims=True)
        a = (a - mu) / jnp.sqrt(va + 1e-5) * g3[...] + be3[...]
        out[bi] = jnp.max(a.reshape(NQ, K, a.shape[-1]), axis=1)


def _mlp(rel, w1t, b1, g1, be1, w2t, b2, g2, be2, w3t, b3, g3, be3):
    return pl.pallas_call(
        _mlp_body,
        out_shape=jax.ShapeDtypeStruct((B, NQ, 384), jnp.float32),
    )(rel, w1t, b1.reshape(1, -1), g1.reshape(1, -1), be1.reshape(1, -1),
      w2t, b2.reshape(1, -1), g2.reshape(1, -1), be2.reshape(1, -1),
      w3t, b3.reshape(1, -1), g3.reshape(1, -1), be3.reshape(1, -1))


def kernel(xyz, features, batch, W1, b1, g1, be1, W2, b2, g2, be2,
           W3, b3, g3, be3):
    del features, batch
    pts = xyz.T.reshape(-1)
    centers = xyz.reshape(B, NPER, 3)[:, ::STEP, :]
    ctr = centers.reshape(NQTOT, 3).T.reshape(-1)
    relflat = _knn(pts, ctr)
    rel = relflat.reshape(3, NQTOT * K).T.reshape(B, NQ * K, 3)
    tokens = _mlp(rel, W1.T, b1, g1, be1, W2.T, b2, g2, be2, W3.T, b3, g3, be3)
    return tokens, centers


# speedup vs baseline: 1.2038x; 1.2038x over previous
import functools

import numpy as np
import jax
import jax.numpy as jnp
from jax import lax
from jax.experimental import pallas as pl
from jax.experimental.pallas import tpu as pltpu
from jax.experimental.pallas import tpu_sc as plsc

B = 8
NPER = 32768
NQ = 64
K = 32
STEP = NPER // NQ
NTOT = B * NPER
NQTOT = B * NQ
NTILES = 32
QPT = NQTOT // NTILES
TPB = NTILES // B
NCHUNK = NPER // 16
INF = np.float32(3.4e38)


def _lex_lt(ka, ia, kb, ib):
    return (ka < kb) | ((ka == kb) & (ia < ib))


FCAP = 48


def _knn_body(pts, ctr, out, xs, ys, zs, cbuf, bufd, bufi, outv,
              fifod, fifoi, curs, thr):
    cid = lax.axis_index("c")
    sid = lax.axis_index("s")
    wid = sid * 2 + cid
    bi = wid // TPB
    qoff = (wid % TPB) * QPT
    base = bi * NPER

    pltpu.sync_copy(pts.at[pl.ds(base, NPER)], xs)
    pltpu.sync_copy(pts.at[pl.ds(NTOT + base, NPER)], ys)
    pltpu.sync_copy(pts.at[pl.ds(2 * NTOT + base, NPER)], zs)
    qbase = wid * QPT
    for c in range(3):
        pltpu.sync_copy(ctr.at[pl.ds(c * NQTOT + qbase, QPT)],
                        cbuf.at[pl.ds(c * QPT, QPT)])

    inf16 = jnp.full((16,), INF, jnp.float32)
    zero16 = jnp.zeros((16,), jnp.int32)
    for q in range(QPT):
        curs[q] = 0
        thr[q] = INF
        for h in range(2):
            bufd[pl.ds(q * K + h * 16, 16)] = inf16
            bufi[pl.ds(q * K + h * 16, 16)] = zero16

    cxv = cbuf[pl.ds(0 * QPT, 16)]
    cyv = cbuf[pl.ds(1 * QPT, 16)]
    czv = cbuf[pl.ds(2 * QPT, 16)]
    cxs = [cxv[q] for q in range(QPT)]
    cys = [cyv[q] for q in range(QPT)]
    czs = [czv[q] for q in range(QPT)]

    iota16 = lax.iota(jnp.int32, 16)

    def _merge(q, dm, ii):
        snew, inew = plsc.sort_key_val(dm, ii)
        b0d = bufd[pl.ds(q * K, 16)]
        b1d = bufd[pl.ds(q * K + 16, 16)]
        b0i = bufi[pl.ds(q * K, 16)]
        b1i = bufi[pl.ds(q * K + 16, 16)]
        rb1d = lax.rev(b1d, (0,))
        rb1i = lax.rev(b1i, (0,))
        lt = _lex_lt(snew, inew, rb1d, rb1i)
        ld = jnp.where(lt, snew, rb1d)
        li = jnp.where(lt, inew, rb1i)
        lsd, lsi = plsc.sort_key_val(ld, li)
        rld = lax.rev(lsd, (0,))
        rli = lax.rev(lsi, (0,))
        lt2 = _lex_lt(b0d, b0i, rld, rli)
        lod = jnp.where(lt2, b0d, rld)
        loi = jnp.where(lt2, b0i, rli)
        hid = jnp.where(lt2, rld, b0d)
        hii = jnp.where(lt2, rli, b0i)
        nb0d, nb0i = plsc.sort_key_val(lod, loi)
        nb1d, nb1i = plsc.sort_key_val(hid, hii)
        bufd[pl.ds(q * K, 16)] = nb0d
        bufd[pl.ds(q * K + 16, 16)] = nb1d
        bufi[pl.ds(q * K, 16)] = nb0i
        bufi[pl.ds(q * K + 16, 16)] = nb1i
        thr[q] = nb1d[15]

    def _chunk(ci, carry):
        b16 = ci * 16
        px = xs[pl.ds(b16, 16)]
        py = ys[pl.ds(b16, 16)]
        pz = zs[pl.ds(b16, 16)]
        sqs = []
        masks = []
        for q in range(QPT):
            dx = px - cxs[q]
            dy = py - cys[q]
            dz = pz - czs[q]
            sq = dx * dx + dy * dy + dz * dz
            sqs.append(sq)
            masks.append(sq < thr[q])
        anym = masks[0]
        for q in range(1, QPT):
            anym = anym | masks[q]

        @pl.when(plsc.all_reduce_population_count(anym)[0] > 0)
        def _():
            iv = b16 + iota16
            for q in range(QPT):
                m = masks[q]
                cnt = plsc.all_reduce_population_count(m)[0]

                @pl.when(cnt > 0)
                def _(q=q, m=m, cnt=cnt):
                    qb = q * FCAP
                    cur = curs[q]
                    plsc.store_compressed(
                        fifod.at[pl.ds(qb + cur, 16)], sqs[q], mask=m)
                    plsc.store_compressed(
                        fifoi.at[pl.ds(qb + cur, 16)], iv, mask=m)
                    ncur = cur + cnt
                    curs[q] = ncur

                    @pl.when(ncur >= 16)
                    def _():
                        f0d = fifod[pl.ds(qb, 16)]
                        f0i = fifoi[pl.ds(qb, 16)]
                        _merge(q, f0d, f0i)
                        fifod[pl.ds(qb, 16)] = fifod[pl.ds(qb + 16, 16)]
                        fifoi[pl.ds(qb, 16)] = fifoi[pl.ds(qb + 16, 16)]
                        curs[q] = ncur - 16

        return carry

    lax.fori_loop(0, NCHUNK, _chunk, 0)

    for q in range(QPT):
        cur = curs[q]
        f0d = fifod[pl.ds(q * FCAP, 16)]
        f0i = fifoi[pl.ds(q * FCAP, 16)]
        dm = jnp.where(iota16 < cur, f0d, INF)
        _merge(q, dm, f0i)

    for q in range(QPT):
        for h in range(2):
            ii = bufi[pl.ds(q * K + h * 16, 16)]
            xg = plsc.load_gather(xs, [ii]) - cxs[q]
            yg = plsc.load_gather(ys, [ii]) - cys[q]
            zg = plsc.load_gather(zs, [ii]) - czs[q]
            outv[pl.ds(0 * QPT * K + q * K + h * 16, 16)] = xg
            outv[pl.ds(1 * QPT * K + q * K + h * 16, 16)] = yg
            outv[pl.ds(2 * QPT * K + q * K + h * 16, 16)] = zg
    obase = wid * QPT * K
    for c in range(3):
        pltpu.sync_copy(outv.at[pl.ds(c * QPT * K, QPT * K)],
                        out.at[pl.ds(c * NQTOT * K + obase, QPT * K)])


@functools.cache
def _knn_kernel():
    return pl.kernel(
        _knn_body,
        out_type=jax.ShapeDtypeStruct((3 * NQTOT * K,), jnp.float32),
        mesh=plsc.VectorSubcoreMesh(core_axis_name="c", subcore_axis_name="s"),
        compiler_params=pltpu.CompilerParams(needs_layout_passes=False),
        scratch_types=[
            pltpu.VMEM((NPER,), jnp.float32),
            pltpu.VMEM((NPER,), jnp.float32),
            pltpu.VMEM((NPER,), jnp.float32),
            pltpu.VMEM((3 * QPT,), jnp.float32),
            pltpu.VMEM((QPT * K,), jnp.float32),
            pltpu.VMEM((QPT * K,), jnp.int32),
            pltpu.VMEM((3 * QPT * K,), jnp.float32),
            pltpu.VMEM((QPT * FCAP,), jnp.float32),
            pltpu.VMEM((QPT * FCAP,), jnp.int32),
            pltpu.SMEM((QPT,), jnp.int32),
            pltpu.SMEM((QPT,), jnp.float32),
        ],
    )


def _knn(pts, ctr):
    return _knn_kernel()(pts, ctr)


def _gelu(x):
    return 0.5 * x * (1.0 + lax.erf(x * jnp.float32(0.7071067811865476)))


def _mlp_body(rel, w1, b1, g1, be1, w2, b2, g2, be2, w3, b3, g3, be3, out):
    for bi in range(B):
        x = rel[bi]
        a = jnp.dot(x, w1[...], preferred_element_type=jnp.float32) + b1[...]
        mu = jnp.mean(a, axis=0, keepdims=True)
        va = jnp.mean((a - mu) * (a - mu), axis=0, keepdims=True)
        a = (a - mu) / jnp.sqrt(va + 1e-5) * g1[...] + be1[...]
        a = _gelu(a)
        a = jnp.dot(a, w2[...], preferred_element_type=jnp.float32) + b2[...]
        mu = jnp.mean(a, axis=0, keepdims=True)
        va = jnp.mean((a - mu) * (a - mu), axis=0, keepdims=True)
        a = (a - mu) / jnp.sqrt(va + 1e-5) * g2[...] + be2[...]
        a = _gelu(a)
        a = jnp.dot(a, w3[...], preferred_element_type=jnp.float32) + b3[...]
        mu = jnp.mean(a, axis=0, keepdims=True)
        va = jnp.mean((a - mu) * (a - mu), axis=0, keepdims=True)
        a = (a - mu) / jnp.sqrt(va + 1e-5) * g3[...] + be3[...]
        out[bi] = jnp.max(a.reshape(NQ, K, a.shape[-1]), axis=1)


def _mlp(rel, w1t, b1, g1, be1, w2t, b2, g2, be2, w3t, b3, g3, be3):
    return pl.pallas_call(
        _mlp_body,
        out_shape=jax.ShapeDtypeStruct((B, NQ, 384), jnp.float32),
    )(rel, w1t, b1.reshape(1, -1), g1.reshape(1, -1), be1.reshape(1, -1),
      w2t, b2.reshape(1, -1), g2.reshape(1, -1), be2.reshape(1, -1),
      w3t, b3.reshape(1, -1), g3.reshape(1, -1), be3.reshape(1, -1))


def kernel(xyz, features, batch, W1, b1, g1, be1, W2, b2, g2, be2,
           W3, b3, g3, be3):
    del features, batch
    pts = xyz.T.reshape(-1)
    centers = xyz.reshape(B, NPER, 3)[:, ::STEP, :]
    ctr = centers.reshape(NQTOT, 3).T.reshape(-1)
    relflat = _knn(pts, ctr)
    rel = relflat.reshape(3, NQTOT * K).T.reshape(B, NQ * K, 3)
    tokens = _mlp(rel, W1.T, b1, g1, be1, W2.T, b2, g2, be2, W3.T, b3, g3, be3)
    return tokens, centers


# speedup vs baseline: 1.7438x; 1.4485x over previous
import functools

import numpy as np
import jax
import jax.numpy as jnp
from jax import lax
from jax.experimental import pallas as pl
from jax.experimental.pallas import tpu as pltpu
from jax.experimental.pallas import tpu_sc as plsc

B = 8
NPER = 32768
NQ = 64
K = 32
STEP = NPER // NQ
NTOT = B * NPER
NQTOT = B * NQ
NTILES = 32
QPT = NQTOT // NTILES
TPB = NTILES // B
NCHUNK = NPER // 16
INF = np.float32(3.4e38)


def _lex_lt(ka, ia, kb, ib):
    return (ka < kb) | ((ka == kb) & (ia < ib))


FCAP = 48


def _knn_body(pts, ctr, out, xs, ys, zs, cbuf, bufd, bufi, outv,
              fifod, fifoi, curs, thr):
    cid = lax.axis_index("c")
    sid = lax.axis_index("s")
    wid = sid * 2 + cid
    bi = wid // TPB
    qoff = (wid % TPB) * QPT
    base = bi * NPER

    pltpu.sync_copy(pts.at[pl.ds(base, NPER)], xs)
    pltpu.sync_copy(pts.at[pl.ds(NTOT + base, NPER)], ys)
    pltpu.sync_copy(pts.at[pl.ds(2 * NTOT + base, NPER)], zs)
    qbase = wid * QPT
    for c in range(3):
        pltpu.sync_copy(ctr.at[pl.ds(c * NQTOT + qbase, QPT)],
                        cbuf.at[pl.ds(c * QPT, QPT)])

    inf16 = jnp.full((16,), INF, jnp.float32)
    zero16 = jnp.zeros((16,), jnp.int32)
    for q in range(QPT):
        curs[q] = 0
        thr[pl.ds(q * 16, 16)] = inf16
        for h in range(2):
            bufd[pl.ds(q * K + h * 16, 16)] = inf16
            bufi[pl.ds(q * K + h * 16, 16)] = zero16

    cxv = cbuf[pl.ds(0 * QPT, 16)]
    cyv = cbuf[pl.ds(1 * QPT, 16)]
    czv = cbuf[pl.ds(2 * QPT, 16)]
    cxs = [cxv[q] for q in range(QPT)]
    cys = [cyv[q] for q in range(QPT)]
    czs = [czv[q] for q in range(QPT)]

    iota16 = lax.iota(jnp.int32, 16)

    def _merge(q, dm, ii):
        snew, inew = plsc.sort_key_val(dm, ii)
        b0d = bufd[pl.ds(q * K, 16)]
        b1d = bufd[pl.ds(q * K + 16, 16)]
        b0i = bufi[pl.ds(q * K, 16)]
        b1i = bufi[pl.ds(q * K + 16, 16)]
        rb1d = lax.rev(b1d, (0,))
        rb1i = lax.rev(b1i, (0,))
        lt = _lex_lt(snew, inew, rb1d, rb1i)
        ld = jnp.where(lt, snew, rb1d)
        li = jnp.where(lt, inew, rb1i)
        lsd, lsi = plsc.sort_key_val(ld, li)
        rld = lax.rev(lsd, (0,))
        rli = lax.rev(lsi, (0,))
        lt2 = _lex_lt(b0d, b0i, rld, rli)
        lod = jnp.where(lt2, b0d, rld)
        loi = jnp.where(lt2, b0i, rli)
        hid = jnp.where(lt2, rld, b0d)
        hii = jnp.where(lt2, rli, b0i)
        nb0d, nb0i = plsc.sort_key_val(lod, loi)
        nb1d, nb1i = plsc.sort_key_val(hid, hii)
        bufd[pl.ds(q * K, 16)] = nb0d
        bufd[pl.ds(q * K + 16, 16)] = nb1d
        bufi[pl.ds(q * K, 16)] = nb0i
        bufi[pl.ds(q * K + 16, 16)] = nb1i
        thr[pl.ds(q * 16, 16)] = jnp.full((16,), nb1d[15], jnp.float32)

    GQ = 8
    for g in range(QPT // GQ):
        qg = [g * GQ + i for i in range(GQ)]
        cxb = [jnp.full((16,), cxs[q], jnp.float32) for q in qg]
        cyb = [jnp.full((16,), cys[q], jnp.float32) for q in qg]
        czb = [jnp.full((16,), czs[q], jnp.float32) for q in qg]

        def _chunk(ci, carry, qg=qg, cxb=cxb, cyb=cyb, czb=czb):
            b16 = ci * 16
            px = xs[pl.ds(b16, 16)]
            py = ys[pl.ds(b16, 16)]
            pz = zs[pl.ds(b16, 16)]
            sqs = []
            masks = []
            for i, q in enumerate(qg):
                dx = px - cxb[i]
                dy = py - cyb[i]
                dz = pz - czb[i]
                sq = dx * dx + dy * dy + dz * dz
                tv = thr[pl.ds(q * 16, 16)]
                sqs.append(sq)
                masks.append(sq < tv)
            ms = list(masks)
            while len(ms) > 1:
                ms = [ms[i] | ms[i + 1] for i in range(0, len(ms) - 1, 2)] \
                    + ([ms[-1]] if len(ms) % 2 else [])
            anym = ms[0]

            @pl.when(jnp.any(anym))
            def _():
                iv = b16 + iota16
                for i, q in enumerate(qg):
                    m = masks[i]

                    @pl.when(jnp.any(m))
                    def _(q=q, i=i, m=m):
                        cnt = plsc.all_reduce_population_count(m)[0]
                        qb = q * FCAP
                        cur = curs[q]
                        plsc.store_compressed(
                            fifod.at[pl.ds(qb + cur, 16)], sqs[i], mask=m)
                        plsc.store_compressed(
                            fifoi.at[pl.ds(qb + cur, 16)], iv, mask=m)
                        ncur = cur + cnt
                        curs[q] = ncur

                        @pl.when(ncur >= 16)
                        def _():
                            f0d = fifod[pl.ds(qb, 16)]
                            f0i = fifoi[pl.ds(qb, 16)]
                            _merge(q, f0d, f0i)
                            fifod[pl.ds(qb, 16)] = fifod[pl.ds(qb + 16, 16)]
                            fifoi[pl.ds(qb, 16)] = fifoi[pl.ds(qb + 16, 16)]
                            curs[q] = ncur - 16

            return carry

        lax.fori_loop(0, NCHUNK, _chunk, 0)

    for q in range(QPT):
        cur = curs[q]
        f0d = fifod[pl.ds(q * FCAP, 16)]
        f0i = fifoi[pl.ds(q * FCAP, 16)]
        dm = jnp.where(iota16 < cur, f0d, INF)
        _merge(q, dm, f0i)

    for q in range(QPT):
        for h in range(2):
            ii = bufi[pl.ds(q * K + h * 16, 16)]
            xg = plsc.load_gather(xs, [ii]) - cxs[q]
            yg = plsc.load_gather(ys, [ii]) - cys[q]
            zg = plsc.load_gather(zs, [ii]) - czs[q]
            outv[pl.ds(0 * QPT * K + q * K + h * 16, 16)] = xg
            outv[pl.ds(1 * QPT * K + q * K + h * 16, 16)] = yg
            outv[pl.ds(2 * QPT * K + q * K + h * 16, 16)] = zg
    obase = wid * QPT * K
    for c in range(3):
        pltpu.sync_copy(outv.at[pl.ds(c * QPT * K, QPT * K)],
                        out.at[pl.ds(c * NQTOT * K + obase, QPT * K)])


@functools.cache
def _knn_kernel():
    return pl.kernel(
        _knn_body,
        out_type=jax.ShapeDtypeStruct((3 * NQTOT * K,), jnp.float32),
        mesh=plsc.VectorSubcoreMesh(core_axis_name="c", subcore_axis_name="s"),
        compiler_params=pltpu.CompilerParams(needs_layout_passes=False),
        scratch_types=[
            pltpu.VMEM((NPER,), jnp.float32),
            pltpu.VMEM((NPER,), jnp.float32),
            pltpu.VMEM((NPER,), jnp.float32),
            pltpu.VMEM((3 * QPT,), jnp.float32),
            pltpu.VMEM((QPT * K,), jnp.float32),
            pltpu.VMEM((QPT * K,), jnp.int32),
            pltpu.VMEM((3 * QPT * K,), jnp.float32),
            pltpu.VMEM((QPT * FCAP,), jnp.float32),
            pltpu.VMEM((QPT * FCAP,), jnp.int32),
            pltpu.SMEM((QPT,), jnp.int32),
            pltpu.VMEM((QPT * 16,), jnp.float32),
        ],
    )


def _knn(pts, ctr):
    return _knn_kernel()(pts, ctr)


def _gelu(x):
    return 0.5 * x * (1.0 + lax.erf(x * jnp.float32(0.7071067811865476)))


def _mlp_body(rel, w1, b1, g1, be1, w2, b2, g2, be2, w3, b3, g3, be3, out):
    for bi in range(B):
        x = rel[bi]
        a = jnp.dot(x, w1[...], preferred_element_type=jnp.float32) + b1[...]
        mu = jnp.mean(a, axis=0, keepdims=True)
        va = jnp.mean((a - mu) * (a - mu), axis=0, keepdims=True)
        a = (a - mu) / jnp.sqrt(va + 1e-5) * g1[...] + be1[...]
        a = _gelu(a)
        a = jnp.dot(a, w2[...], preferred_element_type=jnp.float32) + b2[...]
        mu = jnp.mean(a, axis=0, keepdims=True)
        va = jnp.mean((a - mu) * (a - mu), axis=0, keepdims=True)
        a = (a - mu) / jnp.sqrt(va + 1e-5) * g2[...] + be2[...]
        a = _gelu(a)
        a = jnp.dot(a, w3[...], preferred_element_type=jnp.float32) + b3[...]
        mu = jnp.mean(a, axis=0, keepdims=True)
        va = jnp.mean((a - mu) * (a - mu), axis=0, keepdims=True)
        a = (a - mu) / jnp.sqrt(va + 1e-5) * g3[...] + be3[...]
        out[bi] = jnp.max(a.reshape(NQ, K, a.shape[-1]), axis=1)


def _mlp(rel, w1t, b1, g1, be1, w2t, b2, g2, be2, w3t, b3, g3, be3):
    return pl.pallas_call(
        _mlp_body,
        out_shape=jax.ShapeDtypeStruct((B, NQ, 384), jnp.float32),
    )(rel, w1t, b1.reshape(1, -1), g1.reshape(1, -1), be1.reshape(1, -1),
      w2t, b2.reshape(1, -1), g2.reshape(1, -1), be2.reshape(1, -1),
      w3t, b3.reshape(1, -1), g3.reshape(1, -1), be3.reshape(1, -1))


def kernel(xyz, features, batch, W1, b1, g1, be1, W2, b2, g2, be2,
           W3, b3, g3, be3):
    del features, batch
    pts = xyz.T.reshape(-1)
    centers = xyz.reshape(B, NPER, 3)[:, ::STEP, :]
    ctr = centers.reshape(NQTOT, 3).T.reshape(-1)
    relflat = _knn(pts, ctr)
    rel = relflat.reshape(3, NQTOT * K).T.reshape(B, NQ * K, 3)
    tokens = _mlp(rel, W1.T, b1, g1, be1, W2.T, b2, g2, be2, W3.T, b3, g3, be3)
    return tokens, centers


# speedup vs baseline: 1.7600x; 1.0093x over previous
import functools

import numpy as np
import jax
import jax.numpy as jnp
from jax import lax
from jax.experimental import pallas as pl
from jax.experimental.pallas import tpu as pltpu
from jax.experimental.pallas import tpu_sc as plsc

B = 8
NPER = 32768
NQ = 64
K = 32
STEP = NPER // NQ
NTOT = B * NPER
NQTOT = B * NQ
NTILES = 32
QPT = NQTOT // NTILES
TPB = NTILES // B
NCHUNK = NPER // 16
INF = np.float32(3.4e38)


def _lex_lt(ka, ia, kb, ib):
    return (ka < kb) | ((ka == kb) & (ia < ib))


FCAP = 48


def _knn_body(pts, ctr, out, xs, ys, zs, cbuf, bufd, bufi, outv,
              fifod, fifoi, curs, thr):
    cid = lax.axis_index("c")
    sid = lax.axis_index("s")
    wid = sid * 2 + cid
    bi = wid // TPB
    qoff = (wid % TPB) * QPT
    base = bi * NPER

    pltpu.sync_copy(pts.at[pl.ds(base, NPER)], xs)
    pltpu.sync_copy(pts.at[pl.ds(NTOT + base, NPER)], ys)
    pltpu.sync_copy(pts.at[pl.ds(2 * NTOT + base, NPER)], zs)
    qbase = wid * QPT
    for c in range(3):
        pltpu.sync_copy(ctr.at[pl.ds(c * NQTOT + qbase, QPT)],
                        cbuf.at[pl.ds(c * QPT, QPT)])

    inf16 = jnp.full((16,), INF, jnp.float32)
    zero16 = jnp.zeros((16,), jnp.int32)
    for q in range(QPT):
        curs[q] = 0
        thr[pl.ds(q * 16, 16)] = inf16
        for h in range(2):
            bufd[pl.ds(q * K + h * 16, 16)] = inf16
            bufi[pl.ds(q * K + h * 16, 16)] = zero16

    cxv = cbuf[pl.ds(0 * QPT, 16)]
    cyv = cbuf[pl.ds(1 * QPT, 16)]
    czv = cbuf[pl.ds(2 * QPT, 16)]
    cxs = [cxv[q] for q in range(QPT)]
    cys = [cyv[q] for q in range(QPT)]
    czs = [czv[q] for q in range(QPT)]

    iota16 = lax.iota(jnp.int32, 16)

    def _merge(q, dm, ii):
        snew, inew = plsc.sort_key_val(dm, ii)
        b0d = bufd[pl.ds(q * K, 16)]
        b1d = bufd[pl.ds(q * K + 16, 16)]
        b0i = bufi[pl.ds(q * K, 16)]
        b1i = bufi[pl.ds(q * K + 16, 16)]
        rb1d = lax.rev(b1d, (0,))
        rb1i = lax.rev(b1i, (0,))
        lt = _lex_lt(snew, inew, rb1d, rb1i)
        ld = jnp.where(lt, snew, rb1d)
        li = jnp.where(lt, inew, rb1i)
        lsd, lsi = plsc.sort_key_val(ld, li)
        rld = lax.rev(lsd, (0,))
        rli = lax.rev(lsi, (0,))
        lt2 = _lex_lt(b0d, b0i, rld, rli)
        lod = jnp.where(lt2, b0d, rld)
        loi = jnp.where(lt2, b0i, rli)
        hid = jnp.where(lt2, rld, b0d)
        hii = jnp.where(lt2, rli, b0i)
        nb0d, nb0i = plsc.sort_key_val(lod, loi)
        nb1d, nb1i = plsc.sort_key_val(hid, hii)
        bufd[pl.ds(q * K, 16)] = nb0d
        bufd[pl.ds(q * K + 16, 16)] = nb1d
        bufi[pl.ds(q * K, 16)] = nb0i
        bufi[pl.ds(q * K + 16, 16)] = nb1i
        thr[pl.ds(q * 16, 16)] = jnp.full((16,), nb1d[15], jnp.float32)

    GQ = 8
    for g in range(QPT // GQ):
        qg = [g * GQ + i for i in range(GQ)]
        cxb = [jnp.full((16,), cxs[q], jnp.float32) for q in qg]
        cyb = [jnp.full((16,), cys[q], jnp.float32) for q in qg]
        czb = [jnp.full((16,), czs[q], jnp.float32) for q in qg]

        def _chunk(ci, carry, qg=qg, cxb=cxb, cyb=cyb, czb=czb):
            b16 = ci * 16
            px = xs[pl.ds(b16, 16)]
            py = ys[pl.ds(b16, 16)]
            pz = zs[pl.ds(b16, 16)]
            sqs = []
            masks = []
            for i, q in enumerate(qg):
                dx = px - cxb[i]
                dy = py - cyb[i]
                dz = pz - czb[i]
                sq = dx * dx + dy * dy + dz * dz
                tv = thr[pl.ds(q * 16, 16)]
                sqs.append(sq)
                masks.append(sq < tv)
            ms = list(masks)
            while len(ms) > 1:
                ms = [ms[i] | ms[i + 1] for i in range(0, len(ms) - 1, 2)] \
                    + ([ms[-1]] if len(ms) % 2 else [])
            anym = ms[0]

            @pl.when(jnp.any(anym))
            def _():
                iv = b16 + iota16
                for i, q in enumerate(qg):
                    m = masks[i]

                    @pl.when(jnp.any(m))
                    def _(q=q, i=i, m=m):
                        cnt = plsc.all_reduce_population_count(m)[0]
                        qb = q * FCAP
                        cur = curs[q]
                        plsc.store_compressed(
                            fifod.at[pl.ds(qb + cur, 16)], sqs[i], mask=m)
                        plsc.store_compressed(
                            fifoi.at[pl.ds(qb + cur, 16)], iv, mask=m)
                        ncur = cur + cnt
                        curs[q] = ncur

                        @pl.when(ncur >= 16)
                        def _():
                            f0d = fifod[pl.ds(qb, 16)]
                            f0i = fifoi[pl.ds(qb, 16)]
                            _merge(q, f0d, f0i)
                            fifod[pl.ds(qb, 16)] = fifod[pl.ds(qb + 16, 16)]
                            fifoi[pl.ds(qb, 16)] = fifoi[pl.ds(qb + 16, 16)]
                            curs[q] = ncur - 16

            return carry

        lax.fori_loop(0, NCHUNK, _chunk, 0, unroll=2)

    for q in range(QPT):
        cur = curs[q]
        f0d = fifod[pl.ds(q * FCAP, 16)]
        f0i = fifoi[pl.ds(q * FCAP, 16)]
        dm = jnp.where(iota16 < cur, f0d, INF)
        _merge(q, dm, f0i)

    for q in range(QPT):
        for h in range(2):
            ii = bufi[pl.ds(q * K + h * 16, 16)]
            xg = plsc.load_gather(xs, [ii]) - cxs[q]
            yg = plsc.load_gather(ys, [ii]) - cys[q]
            zg = plsc.load_gather(zs, [ii]) - czs[q]
            outv[pl.ds(0 * QPT * K + q * K + h * 16, 16)] = xg
            outv[pl.ds(1 * QPT * K + q * K + h * 16, 16)] = yg
            outv[pl.ds(2 * QPT * K + q * K + h * 16, 16)] = zg
    obase = wid * QPT * K
    for c in range(3):
        pltpu.sync_copy(outv.at[pl.ds(c * QPT * K, QPT * K)],
                        out.at[pl.ds(c * NQTOT * K + obase, QPT * K)])


@functools.cache
def _knn_kernel():
    return pl.kernel(
        _knn_body,
        out_type=jax.ShapeDtypeStruct((3 * NQTOT * K,), jnp.float32),
        mesh=plsc.VectorSubcoreMesh(core_axis_name="c", subcore_axis_name="s"),
        compiler_params=pltpu.CompilerParams(needs_layout_passes=False),
        scratch_types=[
            pltpu.VMEM((NPER,), jnp.float32),
            pltpu.VMEM((NPER,), jnp.float32),
            pltpu.VMEM((NPER,), jnp.float32),
            pltpu.VMEM((3 * QPT,), jnp.float32),
            pltpu.VMEM((QPT * K,), jnp.float32),
            pltpu.VMEM((QPT * K,), jnp.int32),
            pltpu.VMEM((3 * QPT * K,), jnp.float32),
            pltpu.VMEM((QPT * FCAP,), jnp.float32),
            pltpu.VMEM((QPT * FCAP,), jnp.int32),
            pltpu.SMEM((QPT,), jnp.int32),
            pltpu.VMEM((QPT * 16,), jnp.float32),
        ],
    )


def _knn(pts, ctr):
    return _knn_kernel()(pts, ctr)


def _gelu(x):
    return 0.5 * x * (1.0 + lax.erf(x * jnp.float32(0.7071067811865476)))


def _mlp_body(rel, w1, b1, g1, be1, w2, b2, g2, be2, w3, b3, g3, be3, out):
    for bi in range(B):
        x = rel[bi]
        a = jnp.dot(x, w1[...], preferred_element_type=jnp.float32) + b1[...]
        mu = jnp.mean(a, axis=0, keepdims=True)
        va = jnp.mean((a - mu) * (a - mu), axis=0, keepdims=True)
        a = (a - mu) / jnp.sqrt(va + 1e-5) * g1[...] + be1[...]
        a = _gelu(a)
        a = jnp.dot(a, w2[...], preferred_element_type=jnp.float32) + b2[...]
        mu = jnp.mean(a, axis=0, keepdims=True)
        va = jnp.mean((a - mu) * (a - mu), axis=0, keepdims=True)
        a = (a - mu) / jnp.sqrt(va + 1e-5) * g2[...] + be2[...]
        a = _gelu(a)
        a = jnp.dot(a, w3[...], preferred_element_type=jnp.float32) + b3[...]
        mu = jnp.mean(a, axis=0, keepdims=True)
        va = jnp.mean((a - mu) * (a - mu), axis=0, keepdims=True)
        a = (a - mu) / jnp.sqrt(va + 1e-5) * g3[...] + be3[...]
        out[bi] = jnp.max(a.reshape(NQ, K, a.shape[-1]), axis=1)


def _mlp(rel, w1t, b1, g1, be1, w2t, b2, g2, be2, w3t, b3, g3, be3):
    return pl.pallas_call(
        _mlp_body,
        out_shape=jax.ShapeDtypeStruct((B, NQ, 384), jnp.float32),
    )(rel, w1t, b1.reshape(1, -1), g1.reshape(1, -1), be1.reshape(1, -1),
      w2t, b2.reshape(1, -1), g2.reshape(1, -1), be2.reshape(1, -1),
      w3t, b3.reshape(1, -1), g3.reshape(1, -1), be3.reshape(1, -1))


def kernel(xyz, features, batch, W1, b1, g1, be1, W2, b2, g2, be2,
           W3, b3, g3, be3):
    del features, batch
    pts = xyz.T.reshape(-1)
    centers = xyz.reshape(B, NPER, 3)[:, ::STEP, :]
    ctr = centers.reshape(NQTOT, 3).T.reshape(-1)
    relflat = _knn(pts, ctr)
    rel = relflat.reshape(3, NQTOT * K).T.reshape(B, NQ * K, 3)
    tokens = _mlp(rel, W1.T, b1, g1, be1, W2.T, b2, g2, be2, W3.T, b3, g3, be3)
    return tokens, centers


# speedup vs baseline: 3.3946x; 1.9287x over previous
import functools

import numpy as np
import jax
import jax.numpy as jnp
from jax import lax
from jax.experimental import pallas as pl
from jax.experimental.pallas import tpu as pltpu
from jax.experimental.pallas import tpu_sc as plsc

B = 8
NPER = 32768
NQ = 64
K = 32
STEP = NPER // NQ
NTOT = B * NPER
NQTOT = B * NQ
NTILES = 32
QPT = NQTOT // NTILES
TPB = NTILES // B
NCHUNK = NPER // 16
INF = np.float32(3.4e38)


def _lex_lt(ka, ia, kb, ib):
    return (ka < kb) | ((ka == kb) & (ia < ib))


FCAP = 96


def _knn_body(pts, ctr, out, xs, ys, zs, cbuf, bufd, bufi, outv,
              fifod, fifoi, curs, thr):
    cid = lax.axis_index("c")
    sid = lax.axis_index("s")
    wid = sid * 2 + cid
    bi = wid // TPB
    qoff = (wid % TPB) * QPT
    base = bi * NPER

    pltpu.sync_copy(pts.at[pl.ds(base, NPER)], xs)
    pltpu.sync_copy(pts.at[pl.ds(NTOT + base, NPER)], ys)
    pltpu.sync_copy(pts.at[pl.ds(2 * NTOT + base, NPER)], zs)
    qbase = wid * QPT
    for c in range(3):
        pltpu.sync_copy(ctr.at[pl.ds(c * NQTOT + qbase, QPT)],
                        cbuf.at[pl.ds(c * QPT, QPT)])

    inf16 = jnp.full((16,), INF, jnp.float32)
    zero16 = jnp.zeros((16,), jnp.int32)
    for q in range(QPT):
        curs[q] = 0
        thr[pl.ds(q * 16, 16)] = inf16
        for h in range(2):
            bufd[pl.ds(q * K + h * 16, 16)] = inf16
            bufi[pl.ds(q * K + h * 16, 16)] = zero16

    cxv = cbuf[pl.ds(0 * QPT, 16)]
    cyv = cbuf[pl.ds(1 * QPT, 16)]
    czv = cbuf[pl.ds(2 * QPT, 16)]
    cxs = [cxv[q] for q in range(QPT)]
    cys = [cyv[q] for q in range(QPT)]
    czs = [czv[q] for q in range(QPT)]

    iota16 = lax.iota(jnp.int32, 16)

    def _merge(q, dm, ii):
        snew, inew = plsc.sort_key_val(dm, ii)
        b0d = bufd[pl.ds(q * K, 16)]
        b1d = bufd[pl.ds(q * K + 16, 16)]
        b0i = bufi[pl.ds(q * K, 16)]
        b1i = bufi[pl.ds(q * K + 16, 16)]
        rb1d = lax.rev(b1d, (0,))
        rb1i = lax.rev(b1i, (0,))
        lt = _lex_lt(snew, inew, rb1d, rb1i)
        ld = jnp.where(lt, snew, rb1d)
        li = jnp.where(lt, inew, rb1i)
        lsd, lsi = plsc.sort_key_val(ld, li)
        rld = lax.rev(lsd, (0,))
        rli = lax.rev(lsi, (0,))
        lt2 = _lex_lt(b0d, b0i, rld, rli)
        lod = jnp.where(lt2, b0d, rld)
        loi = jnp.where(lt2, b0i, rli)
        hid = jnp.where(lt2, rld, b0d)
        hii = jnp.where(lt2, rli, b0i)
        nb0d, nb0i = plsc.sort_key_val(lod, loi)
        nb1d, nb1i = plsc.sort_key_val(hid, hii)
        bufd[pl.ds(q * K, 16)] = nb0d
        bufd[pl.ds(q * K + 16, 16)] = nb1d
        bufi[pl.ds(q * K, 16)] = nb0i
        bufi[pl.ds(q * K + 16, 16)] = nb1i
        thr[pl.ds(q * 16, 16)] = jnp.full((16,), nb1d[15], jnp.float32)

    def _drain(q, cur):
        qb = q * FCAP

        def _step(c):
            f0d = fifod[pl.ds(qb, 16)]
            f0i = fifoi[pl.ds(qb, 16)]
            _merge(q, f0d, f0i)
            moved = [fifod[pl.ds(qb + 16 * (j + 1), 16)] for j in range(5)]
            movei = [fifoi[pl.ds(qb + 16 * (j + 1), 16)] for j in range(5)]
            for j in range(5):
                fifod[pl.ds(qb + 16 * j, 16)] = moved[j]
                fifoi[pl.ds(qb + 16 * j, 16)] = movei[j]
            return c - 16

        return lax.while_loop(lambda c: c >= 16, _step, cur)

    GQ = 8
    SUB = 4
    for g in range(QPT // GQ):
        qg = [g * GQ + i for i in range(GQ)]
        cxb = [jnp.full((16,), cxs[q], jnp.float32) for q in qg]
        cyb = [jnp.full((16,), cys[q], jnp.float32) for q in qg]
        czb = [jnp.full((16,), czs[q], jnp.float32) for q in qg]

        def _chunk(ci, carry, qg=qg, cxb=cxb, cyb=cyb, czb=czb):
            b64 = ci * (16 * SUB)
            pxs = [xs[pl.ds(b64 + 16 * s, 16)] for s in range(SUB)]
            pys = [ys[pl.ds(b64 + 16 * s, 16)] for s in range(SUB)]
            pzs = [zs[pl.ds(b64 + 16 * s, 16)] for s in range(SUB)]
            sqs = [[None] * SUB for _ in range(GQ)]
            masks = [[None] * SUB for _ in range(GQ)]
            mq = []
            for i, q in enumerate(qg):
                tv = thr[pl.ds(q * 16, 16)]
                for s in range(SUB):
                    dx = pxs[s] - cxb[i]
                    dy = pys[s] - cyb[i]
                    dz = pzs[s] - czb[i]
                    sq = dx * dx + dy * dy + dz * dz
                    sqs[i][s] = sq
                    masks[i][s] = sq < tv
                mq.append((masks[i][0] | masks[i][1])
                          | (masks[i][2] | masks[i][3]))
            ms = list(mq)
            while len(ms) > 1:
                ms = [ms[i] | ms[i + 1] for i in range(0, len(ms) - 1, 2)] \
                    + ([ms[-1]] if len(ms) % 2 else [])
            anym = ms[0]

            @pl.when(jnp.any(anym))
            def _():
                for i, q in enumerate(qg):
                    @pl.when(jnp.any(mq[i]))
                    def _(q=q, i=i):
                        qb = q * FCAP
                        cur = curs[q]
                        for s in range(SUB):
                            m = masks[i][s]
                            cnt = plsc.all_reduce_population_count(m)[0]
                            plsc.store_compressed(
                                fifod.at[pl.ds(qb + cur, 16)],
                                sqs[i][s], mask=m)
                            plsc.store_compressed(
                                fifoi.at[pl.ds(qb + cur, 16)],
                                b64 + 16 * s + iota16, mask=m)
                            cur = cur + cnt
                        curs[q] = _drain(q, cur)

            return carry

        lax.fori_loop(0, NCHUNK // SUB, _chunk, 0)

    for q in range(QPT):
        cur = curs[q]
        f0d = fifod[pl.ds(q * FCAP, 16)]
        f0i = fifoi[pl.ds(q * FCAP, 16)]
        dm = jnp.where(iota16 < cur, f0d, INF)
        _merge(q, dm, f0i)

    for q in range(QPT):
        for h in range(2):
            ii = bufi[pl.ds(q * K + h * 16, 16)]
            xg = plsc.load_gather(xs, [ii]) - cxs[q]
            yg = plsc.load_gather(ys, [ii]) - cys[q]
            zg = plsc.load_gather(zs, [ii]) - czs[q]
            outv[pl.ds(0 * QPT * K + q * K + h * 16, 16)] = xg
            outv[pl.ds(1 * QPT * K + q * K + h * 16, 16)] = yg
            outv[pl.ds(2 * QPT * K + q * K + h * 16, 16)] = zg
    obase = wid * QPT * K
    for c in range(3):
        pltpu.sync_copy(outv.at[pl.ds(c * QPT * K, QPT * K)],
                        out.at[pl.ds(c * NQTOT * K + obase, QPT * K)])


@functools.cache
def _knn_kernel():
    return pl.kernel(
        _knn_body,
        out_type=jax.ShapeDtypeStruct((3 * NQTOT * K,), jnp.float32),
        mesh=plsc.VectorSubcoreMesh(core_axis_name="c", subcore_axis_name="s"),
        compiler_params=pltpu.CompilerParams(needs_layout_passes=False),
        scratch_types=[
            pltpu.VMEM((NPER,), jnp.float32),
            pltpu.VMEM((NPER,), jnp.float32),
            pltpu.VMEM((NPER,), jnp.float32),
            pltpu.VMEM((3 * QPT,), jnp.float32),
            pltpu.VMEM((QPT * K,), jnp.float32),
            pltpu.VMEM((QPT * K,), jnp.int32),
            pltpu.VMEM((3 * QPT * K,), jnp.float32),
            pltpu.VMEM((QPT * FCAP,), jnp.float32),
            pltpu.VMEM((QPT * FCAP,), jnp.int32),
            pltpu.SMEM((QPT,), jnp.int32),
            pltpu.VMEM((QPT * 16,), jnp.float32),
        ],
    )


def _knn(pts, ctr):
    return _knn_kernel()(pts, ctr)


def _gelu(x):
    return 0.5 * x * (1.0 + lax.erf(x * jnp.float32(0.7071067811865476)))


def _mlp_body(rel, w1, b1, g1, be1, w2, b2, g2, be2, w3, b3, g3, be3, out):
    for bi in range(B):
        x = rel[bi]
        a = jnp.dot(x, w1[...], preferred_element_type=jnp.float32) + b1[...]
        mu = jnp.mean(a, axis=0, keepdims=True)
        va = jnp.mean((a - mu) * (a - mu), axis=0, keepdims=True)
        a = (a - mu) / jnp.sqrt(va + 1e-5) * g1[...] + be1[...]
        a = _gelu(a)
        a = jnp.dot(a, w2[...], preferred_element_type=jnp.float32) + b2[...]
        mu = jnp.mean(a, axis=0, keepdims=True)
        va = jnp.mean((a - mu) * (a - mu), axis=0, keepdims=True)
        a = (a - mu) / jnp.sqrt(va + 1e-5) * g2[...] + be2[...]
        a = _gelu(a)
        a = jnp.dot(a, w3[...], preferred_element_type=jnp.float32) + b3[...]
        mu = jnp.mean(a, axis=0, keepdims=True)
        va = jnp.mean((a - mu) * (a - mu), axis=0, keepdims=True)
        a = (a - mu) / jnp.sqrt(va + 1e-5) * g3[...] + be3[...]
        out[bi] = jnp.max(a.reshape(NQ, K, a.shape[-1]), axis=1)


def _mlp(rel, w1t, b1, g1, be1, w2t, b2, g2, be2, w3t, b3, g3, be3):
    return pl.pallas_call(
        _mlp_body,
        out_shape=jax.ShapeDtypeStruct((B, NQ, 384), jnp.float32),
    )(rel, w1t, b1.reshape(1, -1), g1.reshape(1, -1), be1.reshape(1, -1),
      w2t, b2.reshape(1, -1), g2.reshape(1, -1), be2.reshape(1, -1),
      w3t, b3.reshape(1, -1), g3.reshape(1, -1), be3.reshape(1, -1))


def kernel(xyz, features, batch, W1, b1, g1, be1, W2, b2, g2, be2,
           W3, b3, g3, be3):
    del features, batch
    pts = xyz.T.reshape(-1)
    centers = xyz.reshape(B, NPER, 3)[:, ::STEP, :]
    ctr = centers.reshape(NQTOT, 3).T.reshape(-1)
    relflat = _knn(pts, ctr)
    rel = relflat.reshape(3, NQTOT * K).T.reshape(B, NQ * K, 3)
    tokens = _mlp(rel, W1.T, b1, g1, be1, W2.T, b2, g2, be2, W3.T, b3, g3, be3)
    return tokens, centers


# speedup vs baseline: 4.1885x; 1.2339x over previous
import functools

import numpy as np
import jax
import jax.numpy as jnp
from jax import lax
from jax.experimental import pallas as pl
from jax.experimental.pallas import tpu as pltpu
from jax.experimental.pallas import tpu_sc as plsc

B = 8
NPER = 32768
NQ = 64
K = 32
STEP = NPER // NQ
NTOT = B * NPER
NQTOT = B * NQ
NTILES = 32
QPT = NQTOT // NTILES
TPB = NTILES // B
NCHUNK = NPER // 16
INF = np.float32(3.4e38)


def _lex_lt(ka, ia, kb, ib):
    return (ka < kb) | ((ka == kb) & (ia < ib))


FCAP = 96


def _knn_body(pts, ctr, out, xs, ys, zs, cbuf, bufd, bufi, outv,
              fifod, fifoi, curs, thr):
    cid = lax.axis_index("c")
    sid = lax.axis_index("s")
    wid = sid * 2 + cid
    bi = wid // TPB
    qoff = (wid % TPB) * QPT
    base = bi * NPER

    pltpu.sync_copy(pts.at[pl.ds(base, NPER)], xs)
    pltpu.sync_copy(pts.at[pl.ds(NTOT + base, NPER)], ys)
    pltpu.sync_copy(pts.at[pl.ds(2 * NTOT + base, NPER)], zs)
    qbase = wid * QPT
    for c in range(3):
        pltpu.sync_copy(ctr.at[pl.ds(c * NQTOT + qbase, QPT)],
                        cbuf.at[pl.ds(c * QPT, QPT)])

    inf16 = jnp.full((16,), INF, jnp.float32)
    zero16 = jnp.zeros((16,), jnp.int32)
    for q in range(QPT):
        curs[q] = 0
        thr[pl.ds(q * 16, 16)] = inf16
        for h in range(2):
            bufd[pl.ds(q * K + h * 16, 16)] = inf16
            bufi[pl.ds(q * K + h * 16, 16)] = zero16

    cxv = cbuf[pl.ds(0 * QPT, 16)]
    cyv = cbuf[pl.ds(1 * QPT, 16)]
    czv = cbuf[pl.ds(2 * QPT, 16)]
    cxs = [cxv[q] for q in range(QPT)]
    cys = [cyv[q] for q in range(QPT)]
    czs = [czv[q] for q in range(QPT)]

    iota16 = lax.iota(jnp.int32, 16)

    def _merge(q, dm, ii):
        snew, inew = plsc.sort_key_val(dm, ii)
        b0d = bufd[pl.ds(q * K, 16)]
        b1d = bufd[pl.ds(q * K + 16, 16)]
        b0i = bufi[pl.ds(q * K, 16)]
        b1i = bufi[pl.ds(q * K + 16, 16)]
        rb1d = lax.rev(b1d, (0,))
        rb1i = lax.rev(b1i, (0,))
        lt = _lex_lt(snew, inew, rb1d, rb1i)
        ld = jnp.where(lt, snew, rb1d)
        li = jnp.where(lt, inew, rb1i)
        lsd, lsi = plsc.sort_key_val(ld, li)
        rld = lax.rev(lsd, (0,))
        rli = lax.rev(lsi, (0,))
        lt2 = _lex_lt(b0d, b0i, rld, rli)
        lod = jnp.where(lt2, b0d, rld)
        loi = jnp.where(lt2, b0i, rli)
        hid = jnp.where(lt2, rld, b0d)
        hii = jnp.where(lt2, rli, b0i)
        nb0d, nb0i = plsc.sort_key_val(lod, loi)
        nb1d, nb1i = plsc.sort_key_val(hid, hii)
        bufd[pl.ds(q * K, 16)] = nb0d
        bufd[pl.ds(q * K + 16, 16)] = nb1d
        bufi[pl.ds(q * K, 16)] = nb0i
        bufi[pl.ds(q * K + 16, 16)] = nb1i
        thr[pl.ds(q * 16, 16)] = jnp.full((16,), nb1d[15], jnp.float32)

    def _drain(q, cur):
        qb = q * FCAP

        def _step(c):
            f0d = fifod[pl.ds(qb, 16)]
            f0i = fifoi[pl.ds(qb, 16)]
            _merge(q, f0d, f0i)
            moved = [fifod[pl.ds(qb + 16 * (j + 1), 16)] for j in range(5)]
            movei = [fifoi[pl.ds(qb + 16 * (j + 1), 16)] for j in range(5)]
            for j in range(5):
                fifod[pl.ds(qb + 16 * j, 16)] = moved[j]
                fifoi[pl.ds(qb + 16 * j, 16)] = movei[j]
            return c - 16

        return lax.while_loop(lambda c: c >= 16, _step, cur)

    GQ = 8
    SUB = 4
    for g in range(QPT // GQ):
        qg = [g * GQ + i for i in range(GQ)]
        cxb = [jnp.full((16,), cxs[q], jnp.float32) for q in qg]
        cyb = [jnp.full((16,), cys[q], jnp.float32) for q in qg]
        czb = [jnp.full((16,), czs[q], jnp.float32) for q in qg]

        def _chunk(ci, carry, qg=qg, cxb=cxb, cyb=cyb, czb=czb):
            b64 = ci * (16 * SUB)
            pxs = [xs[pl.ds(b64 + 16 * s, 16)] for s in range(SUB)]
            pys = [ys[pl.ds(b64 + 16 * s, 16)] for s in range(SUB)]
            pzs = [zs[pl.ds(b64 + 16 * s, 16)] for s in range(SUB)]
            sqs = [[None] * SUB for _ in range(GQ)]
            masks = [[None] * SUB for _ in range(GQ)]
            mq = []
            for i, q in enumerate(qg):
                tv = thr[pl.ds(q * 16, 16)]
                for s in range(SUB):
                    dx = pxs[s] - cxb[i]
                    dy = pys[s] - cyb[i]
                    dz = pzs[s] - czb[i]
                    sq = dx * dx + dy * dy + dz * dz
                    sqs[i][s] = sq
                    masks[i][s] = sq < tv
                mq.append((masks[i][0] | masks[i][1])
                          | (masks[i][2] | masks[i][3]))
            ms = list(mq)
            while len(ms) > 1:
                ms = [ms[i] | ms[i + 1] for i in range(0, len(ms) - 1, 2)] \
                    + ([ms[-1]] if len(ms) % 2 else [])
            anym = ms[0]

            @pl.when(jnp.any(anym))
            def _():
                for i, q in enumerate(qg):
                    qb = q * FCAP
                    cur = curs[q]
                    for s in range(SUB):
                        m = masks[i][s]
                        cnt = plsc.all_reduce_population_count(m)[0]
                        plsc.store_compressed(
                            fifod.at[pl.ds(qb + cur, 16)],
                            sqs[i][s], mask=m)
                        plsc.store_compressed(
                            fifoi.at[pl.ds(qb + cur, 16)],
                            b64 + 16 * s + iota16, mask=m)
                        cur = cur + cnt
                    curs[q] = _drain(q, cur)

            return carry

        lax.fori_loop(0, NCHUNK // SUB, _chunk, 0)

    for q in range(QPT):
        cur = curs[q]
        f0d = fifod[pl.ds(q * FCAP, 16)]
        f0i = fifoi[pl.ds(q * FCAP, 16)]
        dm = jnp.where(iota16 < cur, f0d, INF)
        _merge(q, dm, f0i)

    for q in range(QPT):
        for h in range(2):
            ii = bufi[pl.ds(q * K + h * 16, 16)]
            xg = plsc.load_gather(xs, [ii]) - cxs[q]
            yg = plsc.load_gather(ys, [ii]) - cys[q]
            zg = plsc.load_gather(zs, [ii]) - czs[q]
            outv[pl.ds(0 * QPT * K + q * K + h * 16, 16)] = xg
            outv[pl.ds(1 * QPT * K + q * K + h * 16, 16)] = yg
            outv[pl.ds(2 * QPT * K + q * K + h * 16, 16)] = zg
    obase = wid * QPT * K
    for c in range(3):
        pltpu.sync_copy(outv.at[pl.ds(c * QPT * K, QPT * K)],
                        out.at[pl.ds(c * NQTOT * K + obase, QPT * K)])


@functools.cache
def _knn_kernel():
    return pl.kernel(
        _knn_body,
        out_type=jax.ShapeDtypeStruct((3 * NQTOT * K,), jnp.float32),
        mesh=plsc.VectorSubcoreMesh(core_axis_name="c", subcore_axis_name="s"),
        compiler_params=pltpu.CompilerParams(needs_layout_passes=False),
        scratch_types=[
            pltpu.VMEM((NPER,), jnp.float32),
            pltpu.VMEM((NPER,), jnp.float32),
            pltpu.VMEM((NPER,), jnp.float32),
            pltpu.VMEM((3 * QPT,), jnp.float32),
            pltpu.VMEM((QPT * K,), jnp.float32),
            pltpu.VMEM((QPT * K,), jnp.int32),
            pltpu.VMEM((3 * QPT * K,), jnp.float32),
            pltpu.VMEM((QPT * FCAP,), jnp.float32),
            pltpu.VMEM((QPT * FCAP,), jnp.int32),
            pltpu.SMEM((QPT,), jnp.int32),
            pltpu.VMEM((QPT * 16,), jnp.float32),
        ],
    )


def _knn(pts, ctr):
    return _knn_kernel()(pts, ctr)


def _gelu(x):
    return 0.5 * x * (1.0 + lax.erf(x * jnp.float32(0.7071067811865476)))


def _mlp_body(rel, w1, b1, g1, be1, w2, b2, g2, be2, w3, b3, g3, be3, out):
    for bi in range(B):
        x = rel[bi]
        a = jnp.dot(x, w1[...], preferred_element_type=jnp.float32) + b1[...]
        mu = jnp.mean(a, axis=0, keepdims=True)
        va = jnp.mean((a - mu) * (a - mu), axis=0, keepdims=True)
        a = (a - mu) / jnp.sqrt(va + 1e-5) * g1[...] + be1[...]
        a = _gelu(a)
        a = jnp.dot(a, w2[...], preferred_element_type=jnp.float32) + b2[...]
        mu = jnp.mean(a, axis=0, keepdims=True)
        va = jnp.mean((a - mu) * (a - mu), axis=0, keepdims=True)
        a = (a - mu) / jnp.sqrt(va + 1e-5) * g2[...] + be2[...]
        a = _gelu(a)
        a = jnp.dot(a, w3[...], preferred_element_type=jnp.float32) + b3[...]
        mu = jnp.mean(a, axis=0, keepdims=True)
        va = jnp.mean((a - mu) * (a - mu), axis=0, keepdims=True)
        a = (a - mu) / jnp.sqrt(va + 1e-5) * g3[...] + be3[...]
        out[bi] = jnp.max(a.reshape(NQ, K, a.shape[-1]), axis=1)


def _mlp(rel, w1t, b1, g1, be1, w2t, b2, g2, be2, w3t, b3, g3, be3):
    return pl.pallas_call(
        _mlp_body,
        out_shape=jax.ShapeDtypeStruct((B, NQ, 384), jnp.float32),
    )(rel, w1t, b1.reshape(1, -1), g1.reshape(1, -1), be1.reshape(1, -1),
      w2t, b2.reshape(1, -1), g2.reshape(1, -1), be2.reshape(1, -1),
      w3t, b3.reshape(1, -1), g3.reshape(1, -1), be3.reshape(1, -1))


def kernel(xyz, features, batch, W1, b1, g1, be1, W2, b2, g2, be2,
           W3, b3, g3, be3):
    del features, batch
    pts = xyz.T.reshape(-1)
    centers = xyz.reshape(B, NPER, 3)[:, ::STEP, :]
    ctr = centers.reshape(NQTOT, 3).T.reshape(-1)
    relflat = _knn(pts, ctr)
    rel = relflat.reshape(3, NQTOT * K).T.reshape(B, NQ * K, 3)
    tokens = _mlp(rel, W1.T, b1, g1, be1, W2.T, b2, g2, be2, W3.T, b3, g3, be3)
    return tokens, centers


# speedup vs baseline: 4.5689x; 1.0908x over previous
import functools

import numpy as np
import jax
import jax.numpy as jnp
from jax import lax
from jax.experimental import pallas as pl
from jax.experimental.pallas import tpu as pltpu
from jax.experimental.pallas import tpu_sc as plsc

B = 8
NPER = 32768
NQ = 64
K = 32
STEP = NPER // NQ
NTOT = B * NPER
NQTOT = B * NQ
NTILES = 32
QPT = NQTOT // NTILES
TPB = NTILES // B
NCHUNK = NPER // 16
INF = np.float32(3.4e38)


def _lex_lt(ka, ia, kb, ib):
    return (ka < kb) | ((ka == kb) & (ia < ib))


FCAP = 96


def _knn_body(pts, ctr, out, xs, ys, zs, cbuf, bufd, bufi, outv,
              fifod, fifoi, curs, thr):
    cid = lax.axis_index("c")
    sid = lax.axis_index("s")
    wid = sid * 2 + cid
    bi = wid // TPB
    qoff = (wid % TPB) * QPT
    base = bi * NPER

    pltpu.sync_copy(pts.at[pl.ds(base, NPER)], xs)
    pltpu.sync_copy(pts.at[pl.ds(NTOT + base, NPER)], ys)
    pltpu.sync_copy(pts.at[pl.ds(2 * NTOT + base, NPER)], zs)
    qbase = wid * QPT
    for c in range(3):
        pltpu.sync_copy(ctr.at[pl.ds(c * NQTOT + qbase, QPT)],
                        cbuf.at[pl.ds(c * QPT, QPT)])

    inf16 = jnp.full((16,), INF, jnp.float32)
    zero16 = jnp.zeros((16,), jnp.int32)
    for q in range(QPT):
        curs[q] = 0
        thr[pl.ds(q * 16, 16)] = inf16
        for h in range(2):
            bufd[pl.ds(q * K + h * 16, 16)] = inf16
            bufi[pl.ds(q * K + h * 16, 16)] = zero16

    cxv = cbuf[pl.ds(0 * QPT, 16)]
    cyv = cbuf[pl.ds(1 * QPT, 16)]
    czv = cbuf[pl.ds(2 * QPT, 16)]
    cxs = [cxv[q] for q in range(QPT)]
    cys = [cyv[q] for q in range(QPT)]
    czs = [czv[q] for q in range(QPT)]

    iota16 = lax.iota(jnp.int32, 16)

    def _merge(q, dm, ii):
        snew, inew = plsc.sort_key_val(dm, ii)
        b0d = bufd[pl.ds(q * K, 16)]
        b1d = bufd[pl.ds(q * K + 16, 16)]
        b0i = bufi[pl.ds(q * K, 16)]
        b1i = bufi[pl.ds(q * K + 16, 16)]
        rb1d = lax.rev(b1d, (0,))
        rb1i = lax.rev(b1i, (0,))
        lt = _lex_lt(snew, inew, rb1d, rb1i)
        ld = jnp.where(lt, snew, rb1d)
        li = jnp.where(lt, inew, rb1i)
        lsd, lsi = plsc.sort_key_val(ld, li)
        rld = lax.rev(lsd, (0,))
        rli = lax.rev(lsi, (0,))
        lt2 = _lex_lt(b0d, b0i, rld, rli)
        lod = jnp.where(lt2, b0d, rld)
        loi = jnp.where(lt2, b0i, rli)
        hid = jnp.where(lt2, rld, b0d)
        hii = jnp.where(lt2, rli, b0i)
        nb0d, nb0i = plsc.sort_key_val(lod, loi)
        nb1d, nb1i = plsc.sort_key_val(hid, hii)
        bufd[pl.ds(q * K, 16)] = nb0d
        bufd[pl.ds(q * K + 16, 16)] = nb1d
        bufi[pl.ds(q * K, 16)] = nb0i
        bufi[pl.ds(q * K + 16, 16)] = nb1i
        thr[pl.ds(q * 16, 16)] = jnp.full((16,), nb1d[15], jnp.float32)

    def _drain(q, cur):
        qb = q * FCAP

        def _step(c):
            f0d = fifod[pl.ds(qb, 16)]
            f0i = fifoi[pl.ds(qb, 16)]
            _merge(q, f0d, f0i)
            moved = [fifod[pl.ds(qb + 16 * (j + 1), 16)] for j in range(5)]
            movei = [fifoi[pl.ds(qb + 16 * (j + 1), 16)] for j in range(5)]
            for j in range(5):
                fifod[pl.ds(qb + 16 * j, 16)] = moved[j]
                fifoi[pl.ds(qb + 16 * j, 16)] = movei[j]
            return c - 16

        return lax.while_loop(lambda c: c >= 16, _step, cur)

    GQ = 8
    SUB = 4
    for g in range(QPT // GQ):
        qg = [g * GQ + i for i in range(GQ)]
        cxb = [jnp.full((16,), cxs[q], jnp.float32) for q in qg]
        cyb = [jnp.full((16,), cys[q], jnp.float32) for q in qg]
        czb = [jnp.full((16,), czs[q], jnp.float32) for q in qg]

        def _chunk(ci, carry, qg=qg, cxb=cxb, cyb=cyb, czb=czb):
            b64 = ci * (16 * SUB)
            pxs = [xs[pl.ds(b64 + 16 * s, 16)] for s in range(SUB)]
            pys = [ys[pl.ds(b64 + 16 * s, 16)] for s in range(SUB)]
            pzs = [zs[pl.ds(b64 + 16 * s, 16)] for s in range(SUB)]
            sqs = [[None] * SUB for _ in range(GQ)]
            masks = [[None] * SUB for _ in range(GQ)]
            for i, q in enumerate(qg):
                tv = thr[pl.ds(q * 16, 16)]
                for s in range(SUB):
                    dx = pxs[s] - cxb[i]
                    dy = pys[s] - cyb[i]
                    dz = pzs[s] - czb[i]
                    sq = dx * dx + dy * dy + dz * dz
                    sqs[i][s] = sq
                    masks[i][s] = sq < tv

            ivs = [b64 + 16 * s + iota16 for s in range(SUB)]
            for i, q in enumerate(qg):
                qb = q * FCAP
                cur = curs[q]
                for s in range(SUB):
                    m = masks[i][s]
                    cnt = plsc.all_reduce_population_count(m)[0]
                    plsc.store_compressed(
                        fifod.at[pl.ds(qb + cur, 16)], sqs[i][s], mask=m)
                    plsc.store_compressed(
                        fifoi.at[pl.ds(qb + cur, 16)], ivs[s], mask=m)
                    cur = cur + cnt
                curs[q] = _drain(q, cur)

            return carry

        lax.fori_loop(0, NCHUNK // SUB, _chunk, 0)

    for q in range(QPT):
        cur = curs[q]
        f0d = fifod[pl.ds(q * FCAP, 16)]
        f0i = fifoi[pl.ds(q * FCAP, 16)]
        dm = jnp.where(iota16 < cur, f0d, INF)
        _merge(q, dm, f0i)

    for q in range(QPT):
        for h in range(2):
            ii = bufi[pl.ds(q * K + h * 16, 16)]
            xg = plsc.load_gather(xs, [ii]) - cxs[q]
            yg = plsc.load_gather(ys, [ii]) - cys[q]
            zg = plsc.load_gather(zs, [ii]) - czs[q]
            outv[pl.ds(0 * QPT * K + q * K + h * 16, 16)] = xg
            outv[pl.ds(1 * QPT * K + q * K + h * 16, 16)] = yg
            outv[pl.ds(2 * QPT * K + q * K + h * 16, 16)] = zg
    obase = wid * QPT * K
    for c in range(3):
        pltpu.sync_copy(outv.at[pl.ds(c * QPT * K, QPT * K)],
                        out.at[pl.ds(c * NQTOT * K + obase, QPT * K)])


@functools.cache
def _knn_kernel():
    return pl.kernel(
        _knn_body,
        out_type=jax.ShapeDtypeStruct((3 * NQTOT * K,), jnp.float32),
        mesh=plsc.VectorSubcoreMesh(core_axis_name="c", subcore_axis_name="s"),
        compiler_params=pltpu.CompilerParams(needs_layout_passes=False),
        scratch_types=[
            pltpu.VMEM((NPER,), jnp.float32),
            pltpu.VMEM((NPER,), jnp.float32),
            pltpu.VMEM((NPER,), jnp.float32),
            pltpu.VMEM((3 * QPT,), jnp.float32),
            pltpu.VMEM((QPT * K,), jnp.float32),
            pltpu.VMEM((QPT * K,), jnp.int32),
            pltpu.VMEM((3 * QPT * K,), jnp.float32),
            pltpu.VMEM((QPT * FCAP,), jnp.float32),
            pltpu.VMEM((QPT * FCAP,), jnp.int32),
            pltpu.SMEM((QPT,), jnp.int32),
            pltpu.VMEM((QPT * 16,), jnp.float32),
        ],
    )


def _knn(pts, ctr):
    return _knn_kernel()(pts, ctr)


def _gelu(x):
    return 0.5 * x * (1.0 + lax.erf(x * jnp.float32(0.7071067811865476)))


def _mlp_body(rel, w1, b1, g1, be1, w2, b2, g2, be2, w3, b3, g3, be3, out):
    for bi in range(B):
        x = rel[bi]
        a = jnp.dot(x, w1[...], preferred_element_type=jnp.float32) + b1[...]
        mu = jnp.mean(a, axis=0, keepdims=True)
        va = jnp.mean((a - mu) * (a - mu), axis=0, keepdims=True)
        a = (a - mu) / jnp.sqrt(va + 1e-5) * g1[...] + be1[...]
        a = _gelu(a)
        a = jnp.dot(a, w2[...], preferred_element_type=jnp.float32) + b2[...]
        mu = jnp.mean(a, axis=0, keepdims=True)
        va = jnp.mean((a - mu) * (a - mu), axis=0, keepdims=True)
        a = (a - mu) / jnp.sqrt(va + 1e-5) * g2[...] + be2[...]
        a = _gelu(a)
        a = jnp.dot(a, w3[...], preferred_element_type=jnp.float32) + b3[...]
        mu = jnp.mean(a, axis=0, keepdims=True)
        va = jnp.mean((a - mu) * (a - mu), axis=0, keepdims=True)
        a = (a - mu) / jnp.sqrt(va + 1e-5) * g3[...] + be3[...]
        out[bi] = jnp.max(a.reshape(NQ, K, a.shape[-1]), axis=1)


def _mlp(rel, w1t, b1, g1, be1, w2t, b2, g2, be2, w3t, b3, g3, be3):
    return pl.pallas_call(
        _mlp_body,
        out_shape=jax.ShapeDtypeStruct((B, NQ, 384), jnp.float32),
    )(rel, w1t, b1.reshape(1, -1), g1.reshape(1, -1), be1.reshape(1, -1),
      w2t, b2.reshape(1, -1), g2.reshape(1, -1), be2.reshape(1, -1),
      w3t, b3.reshape(1, -1), g3.reshape(1, -1), be3.reshape(1, -1))


def kernel(xyz, features, batch, W1, b1, g1, be1, W2, b2, g2, be2,
           W3, b3, g3, be3):
    del features, batch
    pts = xyz.T.reshape(-1)
    centers = xyz.reshape(B, NPER, 3)[:, ::STEP, :]
    ctr = centers.reshape(NQTOT, 3).T.reshape(-1)
    relflat = _knn(pts, ctr)
    rel = relflat.reshape(3, NQTOT * K).T.reshape(B, NQ * K, 3)
    tokens = _mlp(rel, W1.T, b1, g1, be1, W2.T, b2, g2, be2, W3.T, b3, g3, be3)
    return tokens, centers


# speedup vs baseline: 5.4746x; 1.1982x over previous
import functools

import numpy as np
import jax
import jax.numpy as jnp
from jax import lax
from jax.experimental import pallas as pl
from jax.experimental.pallas import tpu as pltpu
from jax.experimental.pallas import tpu_sc as plsc

B = 8
NPER = 32768
NQ = 64
K = 32
STEP = NPER // NQ
NTOT = B * NPER
NQTOT = B * NQ
NTILES = 32
QPT = NQTOT // NTILES
TPB = NTILES // B
NCHUNK = NPER // 16
INF = np.float32(3.4e38)


def _lex_lt(ka, ia, kb, ib):
    return (ka < kb) | ((ka == kb) & (ia < ib))


FCAP = 96


def _knn_body(pts, ctr, out, xs, ys, zs, cbuf, bufd, bufi, outv,
              fifoi, curs, thr):
    cid = lax.axis_index("c")
    sid = lax.axis_index("s")
    wid = sid * 2 + cid
    bi = wid // TPB
    qoff = (wid % TPB) * QPT
    base = bi * NPER

    pltpu.sync_copy(pts.at[pl.ds(base, NPER)], xs)
    pltpu.sync_copy(pts.at[pl.ds(NTOT + base, NPER)], ys)
    pltpu.sync_copy(pts.at[pl.ds(2 * NTOT + base, NPER)], zs)
    qbase = wid * QPT
    for c in range(3):
        pltpu.sync_copy(ctr.at[pl.ds(c * NQTOT + qbase, QPT)],
                        cbuf.at[pl.ds(c * QPT, QPT)])

    inf16 = jnp.full((16,), INF, jnp.float32)
    zero16 = jnp.zeros((16,), jnp.int32)
    for q in range(QPT):
        curs[q] = 0
        thr[pl.ds(q * 16, 16)] = inf16
        for h in range(2):
            bufd[pl.ds(q * K + h * 16, 16)] = inf16
            bufi[pl.ds(q * K + h * 16, 16)] = zero16

    cxv = cbuf[pl.ds(0 * QPT, 16)]
    cyv = cbuf[pl.ds(1 * QPT, 16)]
    czv = cbuf[pl.ds(2 * QPT, 16)]
    cxs = [cxv[q] for q in range(QPT)]
    cys = [cyv[q] for q in range(QPT)]
    czs = [czv[q] for q in range(QPT)]

    iota16 = lax.iota(jnp.int32, 16)

    def _merge(q, dm, ii):
        snew, inew = plsc.sort_key_val(dm, ii)
        b0d = bufd[pl.ds(q * K, 16)]
        b1d = bufd[pl.ds(q * K + 16, 16)]
        b0i = bufi[pl.ds(q * K, 16)]
        b1i = bufi[pl.ds(q * K + 16, 16)]
        rb1d = lax.rev(b1d, (0,))
        rb1i = lax.rev(b1i, (0,))
        lt = _lex_lt(snew, inew, rb1d, rb1i)
        ld = jnp.where(lt, snew, rb1d)
        li = jnp.where(lt, inew, rb1i)
        lsd, lsi = plsc.sort_key_val(ld, li)
        rld = lax.rev(lsd, (0,))
        rli = lax.rev(lsi, (0,))
        lt2 = _lex_lt(b0d, b0i, rld, rli)
        lod = jnp.where(lt2, b0d, rld)
        loi = jnp.where(lt2, b0i, rli)
        hid = jnp.where(lt2, rld, b0d)
        hii = jnp.where(lt2, rli, b0i)
        nb0d, nb0i = plsc.sort_key_val(lod, loi)
        nb1d, nb1i = plsc.sort_key_val(hid, hii)
        bufd[pl.ds(q * K, 16)] = nb0d
        bufd[pl.ds(q * K + 16, 16)] = nb1d
        bufi[pl.ds(q * K, 16)] = nb0i
        bufi[pl.ds(q * K + 16, 16)] = nb1i
        thr[pl.ds(q * 16, 16)] = jnp.full((16,), nb1d[15], jnp.float32)

    def _resq(q, ii):
        xg = plsc.load_gather(xs, [ii])
        yg = plsc.load_gather(ys, [ii])
        zg = plsc.load_gather(zs, [ii])
        dx = xg - cxs[q]
        dy = yg - cys[q]
        dz = zg - czs[q]
        return dx * dx + dy * dy + dz * dz

    def _drain(q, cur):
        qb = q * FCAP

        def _step(c):
            f0i = fifoi[pl.ds(qb, 16)]
            _merge(q, _resq(q, f0i), f0i)
            movei = [fifoi[pl.ds(qb + 16 * (j + 1), 16)] for j in range(5)]
            for j in range(5):
                fifoi[pl.ds(qb + 16 * j, 16)] = movei[j]
            return c - 16

        return lax.while_loop(lambda c: c >= 16, _step, cur)

    GQ = 8
    SUB = 4
    for g in range(QPT // GQ):
        qg = [g * GQ + i for i in range(GQ)]
        cxb = [jnp.full((16,), cxs[q], jnp.float32) for q in qg]
        cyb = [jnp.full((16,), cys[q], jnp.float32) for q in qg]
        czb = [jnp.full((16,), czs[q], jnp.float32) for q in qg]

        def _chunk(ci, carry, qg=qg, cxb=cxb, cyb=cyb, czb=czb):
            b64 = ci * (16 * SUB)
            pxs = [xs[pl.ds(b64 + 16 * s, 16)] for s in range(SUB)]
            pys = [ys[pl.ds(b64 + 16 * s, 16)] for s in range(SUB)]
            pzs = [zs[pl.ds(b64 + 16 * s, 16)] for s in range(SUB)]
            sqs = [[None] * SUB for _ in range(GQ)]
            masks = [[None] * SUB for _ in range(GQ)]
            for i, q in enumerate(qg):
                tv = thr[pl.ds(q * 16, 16)]
                for s in range(SUB):
                    dx = pxs[s] - cxb[i]
                    dy = pys[s] - cyb[i]
                    dz = pzs[s] - czb[i]
                    sq = dx * dx + dy * dy + dz * dz
                    sqs[i][s] = sq
                    masks[i][s] = sq < tv

            ivs = [b64 + 16 * s + iota16 for s in range(SUB)]
            ncurs = []
            for i, q in enumerate(qg):
                qb = q * FCAP
                cur = curs[q]
                for s in range(SUB):
                    m = masks[i][s]
                    cnt = plsc.all_reduce_population_count(m)[0]
                    plsc.store_compressed(
                        fifoi.at[pl.ds(qb + cur, 16)], ivs[s], mask=m)
                    cur = cur + cnt
                curs[q] = cur
                ncurs.append(cur)
            maxc = ncurs[0]
            for i in range(1, GQ):
                maxc = lax.max(maxc, ncurs[i])

            @pl.when(maxc >= 16)
            def _():
                for i, q in enumerate(qg):
                    curs[q] = _drain(q, ncurs[i])

            return carry

        lax.fori_loop(0, NCHUNK // SUB, _chunk, 0)

    for q in range(QPT):
        cur = curs[q]
        f0i = jnp.where(iota16 < cur, fifoi[pl.ds(q * FCAP, 16)], 0)
        dm = jnp.where(iota16 < cur, _resq(q, f0i), INF)
        _merge(q, dm, f0i)

    for q in range(QPT):
        for h in range(2):
            ii = bufi[pl.ds(q * K + h * 16, 16)]
            xg = plsc.load_gather(xs, [ii]) - cxs[q]
            yg = plsc.load_gather(ys, [ii]) - cys[q]
            zg = plsc.load_gather(zs, [ii]) - czs[q]
            outv[pl.ds(0 * QPT * K + q * K + h * 16, 16)] = xg
            outv[pl.ds(1 * QPT * K + q * K + h * 16, 16)] = yg
            outv[pl.ds(2 * QPT * K + q * K + h * 16, 16)] = zg
    obase = wid * QPT * K
    for c in range(3):
        pltpu.sync_copy(outv.at[pl.ds(c * QPT * K, QPT * K)],
                        out.at[pl.ds(c * NQTOT * K + obase, QPT * K)])


@functools.cache
def _knn_kernel():
    return pl.kernel(
        _knn_body,
        out_type=jax.ShapeDtypeStruct((3 * NQTOT * K,), jnp.float32),
        mesh=plsc.VectorSubcoreMesh(core_axis_name="c", subcore_axis_name="s"),
        compiler_params=pltpu.CompilerParams(needs_layout_passes=False),
        scratch_types=[
            pltpu.VMEM((NPER,), jnp.float32),
            pltpu.VMEM((NPER,), jnp.float32),
            pltpu.VMEM((NPER,), jnp.float32),
            pltpu.VMEM((3 * QPT,), jnp.float32),
            pltpu.VMEM((QPT * K,), jnp.float32),
            pltpu.VMEM((QPT * K,), jnp.int32),
            pltpu.VMEM((3 * QPT * K,), jnp.float32),
            pltpu.VMEM((QPT * FCAP,), jnp.int32),
            pltpu.SMEM((QPT,), jnp.int32),
            pltpu.VMEM((QPT * 16,), jnp.float32),
        ],
    )


def _knn(pts, ctr):
    return _knn_kernel()(pts, ctr)


def _gelu(x):
    return 0.5 * x * (1.0 + lax.erf(x * jnp.float32(0.7071067811865476)))


def _mlp_body(rel, w1, b1, g1, be1, w2, b2, g2, be2, w3, b3, g3, be3, out):
    for bi in range(B):
        x = rel[bi]
        a = jnp.dot(x, w1[...], preferred_element_type=jnp.float32) + b1[...]
        mu = jnp.mean(a, axis=0, keepdims=True)
        va = jnp.mean((a - mu) * (a - mu), axis=0, keepdims=True)
        a = (a - mu) / jnp.sqrt(va + 1e-5) * g1[...] + be1[...]
        a = _gelu(a)
        a = jnp.dot(a, w2[...], preferred_element_type=jnp.float32) + b2[...]
        mu = jnp.mean(a, axis=0, keepdims=True)
        va = jnp.mean((a - mu) * (a - mu), axis=0, keepdims=True)
        a = (a - mu) / jnp.sqrt(va + 1e-5) * g2[...] + be2[...]
        a = _gelu(a)
        a = jnp.dot(a, w3[...], preferred_element_type=jnp.float32) + b3[...]
        mu = jnp.mean(a, axis=0, keepdims=True)
        va = jnp.mean((a - mu) * (a - mu), axis=0, keepdims=True)
        a = (a - mu) / jnp.sqrt(va + 1e-5) * g3[...] + be3[...]
        out[bi] = jnp.max(a.reshape(NQ, K, a.shape[-1]), axis=1)


def _mlp(rel, w1t, b1, g1, be1, w2t, b2, g2, be2, w3t, b3, g3, be3):
    return pl.pallas_call(
        _mlp_body,
        out_shape=jax.ShapeDtypeStruct((B, NQ, 384), jnp.float32),
    )(rel, w1t, b1.reshape(1, -1), g1.reshape(1, -1), be1.reshape(1, -1),
      w2t, b2.reshape(1, -1), g2.reshape(1, -1), be2.reshape(1, -1),
      w3t, b3.reshape(1, -1), g3.reshape(1, -1), be3.reshape(1, -1))


def kernel(xyz, features, batch, W1, b1, g1, be1, W2, b2, g2, be2,
           W3, b3, g3, be3):
    del features, batch
    pts = xyz.T.reshape(-1)
    centers = xyz.reshape(B, NPER, 3)[:, ::STEP, :]
    ctr = centers.reshape(NQTOT, 3).T.reshape(-1)
    relflat = _knn(pts, ctr)
    rel = relflat.reshape(3, NQTOT * K).T.reshape(B, NQ * K, 3)
    tokens = _mlp(rel, W1.T, b1, g1, be1, W2.T, b2, g2, be2, W3.T, b3, g3, be3)
    return tokens, centers
